# baseline (speedup 1.0000x reference)
import jax
import jax.numpy as jnp
import numpy as np
from jax.experimental import pallas as pl
from jax.experimental.pallas import tpu as pltpu

B = 2
C_IN = 64
H = 96
W = 320
HEAD_CONV = 256
NUM_CLASS = 3
KDET = 50
C_ROI = C_IN + 2 + NUM_CLASS
HW = H * W


def _nms_body(h_ref, o_ref):
    x = h_ref[...]
    ninf = jnp.float32(-jnp.inf)
    up = jnp.concatenate([x[:, :, 1:, :], jnp.full((B, 3, 1, W), ninf)], axis=2)
    dn = jnp.concatenate([jnp.full((B, 3, 1, W), ninf), x[:, :, :-1, :]], axis=2)
    m1 = jnp.maximum(jnp.maximum(x, up), dn)
    lf = jnp.concatenate([m1[:, :, :, 1:], jnp.full((B, 3, H, 1), ninf)], axis=3)
    rt = jnp.concatenate([jnp.full((B, 3, H, 1), ninf), m1[:, :, :, :-1]], axis=3)
    hmax = jnp.maximum(jnp.maximum(m1, lf), rt)
    o_ref[...] = x * (hmax == x).astype(x.dtype)


def _nms_pallas(h):
    return pl.pallas_call(
        _nms_body,
        out_shape=jax.ShapeDtypeStruct(h.shape, h.dtype),
    )(h)


_K1_TILE = 1280


def _k1_body(x_ref, w1_ref, b1_ref, w2_ref, b2_ref, o_ref):
    a = jnp.dot(w1_ref[...], x_ref[0], preferred_element_type=jnp.float32)
    a = jnp.maximum(a + b1_ref[...], 0.0)
    z = jnp.dot(w2_ref[...], a, preferred_element_type=jnp.float32) + b2_ref[...]
    o_ref[0] = z


def _dense_heads_pallas(XT, W1T, b1, W2T, b2):
    nt = HW // _K1_TILE
    return pl.pallas_call(
        _k1_body,
        grid=(B, nt),
        in_specs=[
            pl.BlockSpec((1, 576, _K1_TILE), lambda b, t: (b, 0, t)),
            pl.BlockSpec((768, 576), lambda b, t: (0, 0)),
            pl.BlockSpec((768, 1), lambda b, t: (0, 0)),
            pl.BlockSpec((8, 768), lambda b, t: (0, 0)),
            pl.BlockSpec((8, 1), lambda b, t: (0, 0)),
        ],
        out_specs=pl.BlockSpec((1, 8, _K1_TILE), lambda b, t: (b, 0, t)),
        out_shape=jax.ShapeDtypeStruct((B, 8, HW), jnp.float32),
    )(XT, W1T, b1, W2T, b2)


def _dense_heads(features, p):
    xp = jnp.pad(features, ((0, 0), (0, 0), (1, 1), (1, 1)))
    taps = [xp[:, :, dy:dy + H, dx:dx + W] for dy in range(3) for dx in range(3)]
    XT = jnp.concatenate(taps, axis=1).reshape(B, 576, HW)
    w1 = jnp.concatenate([p['hm_w1'], p['o2d_w1'], p['s2d_w1']], axis=0)
    W1T = w1.transpose(0, 2, 3, 1).reshape(768, 576)
    b1 = jnp.concatenate([p['hm_b1'], p['o2d_b1'], p['s2d_b1']])[:, None]
    W2T = jnp.zeros((8, 768), jnp.float32)
    W2T = W2T.at[0:3, 0:256].set(p['hm_w2'][:, :, 0, 0])
    W2T = W2T.at[3:5, 256:512].set(p['o2d_w2'][:, :, 0, 0])
    W2T = W2T.at[5:7, 512:768].set(p['s2d_w2'][:, :, 0, 0])
    b2 = jnp.concatenate([p['hm_b2'], p['o2d_b2'], p['s2d_b2'],
                          jnp.zeros((1,), jnp.float32)])[:, None]
    dense = _dense_heads_pallas(XT, W1T, b1, W2T, b2)
    heatmap = dense[:, 0:3, :].reshape(B, 3, H, W)
    offset_2d = dense[:, 3:5, :].reshape(B, 2, H, W)
    size_2d = dense[:, 5:7, :].reshape(B, 2, H, W)
    return heatmap, offset_2d, size_2d


def _conv(x, w, b, pad):
    y = jax.lax.conv_general_dilated(x, w, (1, 1), pad,
                                     dimension_numbers=('NCHW', 'OIHW', 'NCHW'))
    return y + b[None, :, None, None]


def _dense_head(x, p, name):
    h = jax.nn.relu(_conv(x, p[name + '_w1'], p[name + '_b1'], 'SAME'))
    return _conv(h, p[name + '_w2'], p[name + '_b2'], 'VALID')


def _roi_head(x, p, name):
    h = _conv(x, p[name + '_w1'], p[name + '_b1'], 'SAME')
    h = (h - p[name + '_bn_m'][None, :, None, None]) / jnp.sqrt(p[name + '_bn_v'][None, :, None, None] + 1e-5)
    h = h * p[name + '_bn_g'][None, :, None, None] + p[name + '_bn_b'][None, :, None, None]
    h = jax.nn.relu(h)
    h = jnp.mean(h, axis=(2, 3), keepdims=True)
    return _conv(h, p[name + '_w2'], p[name + '_b2'], 'VALID')


def _select_topk(heat, K):
    b, c, hh, ww = heat.shape
    flat = heat.reshape(b, c, hh * ww)
    s_all, i_all = jax.lax.top_k(flat, K)
    scores, inds = jax.lax.top_k(s_all.reshape(b, c * K), K)
    clses = inds // K
    inds_all = jnp.take_along_axis(i_all.reshape(b, c * K), inds, axis=1)
    return scores, inds_all, clses


def _bilinear(img, xs, ys):
    x0 = jnp.floor(xs)
    y0 = jnp.floor(ys)
    wx = xs - x0
    wy = ys - y0
    x0i = jnp.clip(x0.astype(jnp.int32), 0, W - 1)
    x1i = jnp.clip(x0.astype(jnp.int32) + 1, 0, W - 1)
    y0i = jnp.clip(y0.astype(jnp.int32), 0, H - 1)
    y1i = jnp.clip(y0.astype(jnp.int32) + 1, 0, H - 1)
    Ia = img[:, y0i, x0i]
    Ib = img[:, y0i, x1i]
    Ic = img[:, y1i, x0i]
    Id = img[:, y1i, x1i]
    return Ia * (1 - wx) * (1 - wy) + Ib * wx * (1 - wy) + Ic * (1 - wx) * wy + Id * wx * wy


def _roi_align(feat, boxes):
    def one(box):
        bidx = box[0].astype(jnp.int32)
        g = (jnp.arange(7, dtype=jnp.float32) + 0.5) / 7.0
        xs = box[1] + g * (box[3] - box[1])
        ys = box[2] + g * (box[4] - box[2])
        xg, yg = jnp.meshgrid(xs, ys)
        return _bilinear(feat[bidx], xg - 0.5, yg - 0.5)
    return jax.vmap(one)(boxes)


def _project(calib, pts):
    cu = calib[:, 0, 2]
    cv = calib[:, 1, 2]
    fu = calib[:, 0, 0]
    fv = calib[:, 1, 1]
    bx = calib[:, 0, 3] / (-fu)
    by = calib[:, 1, 3] / (-fv)
    x = (pts[:, 0] - cu) * pts[:, 2] / fu + bx
    y = (pts[:, 1] - cv) * pts[:, 2] / fv + by
    return jnp.stack([x, y, pts[:, 2]], -1)


def kernel(features, calib, coord_range, params):
    p = params
    heatmap, offset_2d, size_2d = _dense_heads(features, p)
    hm_nms = _nms_pallas(heatmap)
    scores, inds, clses = _select_topk(hm_nms, KDET)
    xg, yg = jnp.meshgrid(jnp.arange(W, dtype=jnp.float32), jnp.arange(H, dtype=jnp.float32))
    coord_map = jnp.broadcast_to(jnp.stack([xg, yg], 0)[None], (B, 2, H, W))
    center = coord_map + offset_2d
    bmaps = jnp.concatenate([center - size_2d / 2.0, center + size_2d / 2.0], 1)
    bids = jnp.broadcast_to(jnp.arange(B, dtype=jnp.float32)[:, None, None, None], (B, 1, H, W))
    bmaps = jnp.concatenate([bids, bmaps], 1)
    bm = bmaps.reshape(B, 5, H * W).transpose(0, 2, 1)
    box = jnp.take_along_axis(bm, inds[:, :, None], axis=1).reshape(B * KDET, 5)
    cls_ids = clses.reshape(B * KDET)
    roi_feat = _roi_align(features, box)
    bidx = box[:, 0].astype(jnp.int32)
    cr = coord_range[bidx]
    sx = cr[:, 1, 0] - cr[:, 0, 0]
    ox = cr[:, 0, 0]
    sy = cr[:, 1, 1] - cr[:, 0, 1]
    oy = cr[:, 0, 1]
    box_s = jnp.stack([box[:, 0], box[:, 1] / W * sx + ox, box[:, 2] / H * sy + oy,
                       box[:, 3] / W * sx + ox, box[:, 4] / H * sy + oy], -1)
    roi_calib = calib[bidx]
    N = B * KDET
    ones = jnp.ones((N, 1), dtype=jnp.float32)
    p1 = _project(roi_calib, jnp.concatenate([box_s[:, 1:3], ones], -1))[:, :2]
    p2 = _project(roi_calib, jnp.concatenate([box_s[:, 3:5], ones], -1))[:, :2]
    cic = jnp.concatenate([box_s[:, 0:1], p1, p2], -1)
    t = jnp.arange(7, dtype=jnp.float32) / 6.0
    cx = cic[:, 1:2] + t[None, :] * (cic[:, 3:4] - cic[:, 1:2])
    cy = cic[:, 2:3] + t[None, :] * (cic[:, 4:5] - cic[:, 2:3])
    coord_maps = jnp.concatenate([
        jnp.broadcast_to(cx[:, None, None, :], (N, 1, 7, 7)),
        jnp.broadcast_to(cy[:, None, :, None], (N, 1, 7, 7))], 1)
    cls_hot = jax.nn.one_hot(cls_ids, NUM_CLASS, dtype=jnp.float32)
    roi_in = jnp.concatenate([roi_feat, coord_maps,
                              jnp.broadcast_to(cls_hot[:, :, None, None], (N, NUM_CLASS, 7, 7))], 1)
    box2d_h = jnp.clip(box_s[:, 4] - box_s[:, 2], 1.0, None)
    s3d = _roi_head(roi_in, p, 's3d')[:, :, 0, 0]
    h3d_log_std = s3d[:, 3:4]
    size_3d = p['mean_size'][cls_ids] + s3d[:, :3]
    depth_geo = size_3d[:, 0] / box2d_h * roi_calib[:, 0, 0]
    dnet = _roi_head(roi_in, p, 'dep')[:, :, 0, 0]
    dgls = (h3d_log_std[:, 0] + 2.0 * (jnp.log(roi_calib[:, 0, 0]) - jnp.log(box2d_h)))[:, None]
    dnls = jax.nn.logsumexp(jnp.concatenate([dnet[:, 1:2], dgls], -1), axis=-1, keepdims=True)
    depth = jnp.concatenate([1.0 / (jax.nn.sigmoid(dnet[:, 0:1]) + 1e-6) - 1.0 + depth_geo[:, None], dnls], -1)
    heading = _roi_head(roi_in, p, 'hd')[:, :, 0, 0]
    offset_3d = _roi_head(roi_in, p, 'o3d')[:, :, 0, 0]
    return heatmap, offset_2d, size_2d, heading, depth, offset_3d, size_3d


# speedup vs baseline: 1.2541x; 1.2541x over previous
import jax
import jax.numpy as jnp
import numpy as np
from jax.experimental import pallas as pl
from jax.experimental.pallas import tpu as pltpu

B = 2
C_IN = 64
H = 96
W = 320
HEAD_CONV = 256
NUM_CLASS = 3
KDET = 50
C_ROI = C_IN + 2 + NUM_CLASS
HW = H * W


def _nms_body(h_ref, o_ref):
    x = h_ref[...]
    ninf = jnp.float32(-jnp.inf)
    up = jnp.concatenate([x[:, :, 1:, :], jnp.full((B, 3, 1, W), ninf)], axis=2)
    dn = jnp.concatenate([jnp.full((B, 3, 1, W), ninf), x[:, :, :-1, :]], axis=2)
    m1 = jnp.maximum(jnp.maximum(x, up), dn)
    lf = jnp.concatenate([m1[:, :, :, 1:], jnp.full((B, 3, H, 1), ninf)], axis=3)
    rt = jnp.concatenate([jnp.full((B, 3, H, 1), ninf), m1[:, :, :, :-1]], axis=3)
    hmax = jnp.maximum(jnp.maximum(m1, lf), rt)
    o_ref[...] = x * (hmax == x).astype(x.dtype)


def _nms_pallas(h):
    return pl.pallas_call(
        _nms_body,
        out_shape=jax.ShapeDtypeStruct(h.shape, h.dtype),
    )(h)


_K1_ROWS = 8
_WP = 384
_XF_COLS = (H + 2) * _WP + 128


def _k1_body(x_ref, w1_ref, b1_ref, w2_ref, b2_ref, o_ref):
    t = pl.program_id(1)
    wide = _K1_ROWS * _WP
    slabs = []
    for dy in range(3):
        start = pl.multiple_of((t * _K1_ROWS + dy) * _WP, 128)
        slab = x_ref[0, :, pl.ds(start, wide + 128)]
        for dx in range(3):
            slabs.append(slab[:, dx:dx + wide])
    rhs = jnp.concatenate(slabs, axis=0)
    a = jnp.dot(w1_ref[...], rhs, preferred_element_type=jnp.float32)
    a = jnp.maximum(a + b1_ref[...], 0.0)
    z = jnp.dot(w2_ref[...], a, preferred_element_type=jnp.float32) + b2_ref[...]
    for r in range(_K1_ROWS):
        o_ref[0, :, r * W:(r + 1) * W] = z[:, r * _WP:r * _WP + W]


def _dense_heads_pallas(XF, W1T, b1, W2T, b2):
    return pl.pallas_call(
        _k1_body,
        grid=(B, H // _K1_ROWS),
        in_specs=[
            pl.BlockSpec((1, 64, _XF_COLS), lambda b, t: (b, 0, 0)),
            pl.BlockSpec((768, 576), lambda b, t: (0, 0)),
            pl.BlockSpec((768, 1), lambda b, t: (0, 0)),
            pl.BlockSpec((8, 768), lambda b, t: (0, 0)),
            pl.BlockSpec((8, 1), lambda b, t: (0, 0)),
        ],
        out_specs=pl.BlockSpec((1, 8, _K1_ROWS * W), lambda b, t: (b, 0, t)),
        out_shape=jax.ShapeDtypeStruct((B, 8, HW), jnp.float32),
    )(XF, W1T, b1, W2T, b2)


def _dense_heads(features, p):
    xp = jnp.pad(features, ((0, 0), (0, 0), (1, 1), (1, 1), ))
    xp = jnp.pad(xp, ((0, 0), (0, 0), (0, 0), (0, _WP - W - 2)))
    XF = jnp.pad(xp.reshape(B, 64, (H + 2) * _WP), ((0, 0), (0, 0), (0, 128)))
    w1 = jnp.concatenate([p['hm_w1'], p['o2d_w1'], p['s2d_w1']], axis=0)
    W1T = w1.transpose(0, 2, 3, 1).reshape(768, 576)
    b1 = jnp.concatenate([p['hm_b1'], p['o2d_b1'], p['s2d_b1']])[:, None]
    W2T = jnp.zeros((8, 768), jnp.float32)
    W2T = W2T.at[0:3, 0:256].set(p['hm_w2'][:, :, 0, 0])
    W2T = W2T.at[3:5, 256:512].set(p['o2d_w2'][:, :, 0, 0])
    W2T = W2T.at[5:7, 512:768].set(p['s2d_w2'][:, :, 0, 0])
    b2 = jnp.concatenate([p['hm_b2'], p['o2d_b2'], p['s2d_b2'],
                          jnp.zeros((1,), jnp.float32)])[:, None]
    dense = _dense_heads_pallas(XF, W1T, b1, W2T, b2)
    heatmap = dense[:, 0:3, :].reshape(B, 3, H, W)
    offset_2d = dense[:, 3:5, :].reshape(B, 2, H, W)
    size_2d = dense[:, 5:7, :].reshape(B, 2, H, W)
    return heatmap, offset_2d, size_2d


def _conv(x, w, b, pad):
    y = jax.lax.conv_general_dilated(x, w, (1, 1), pad,
                                     dimension_numbers=('NCHW', 'OIHW', 'NCHW'))
    return y + b[None, :, None, None]


def _dense_head(x, p, name):
    h = jax.nn.relu(_conv(x, p[name + '_w1'], p[name + '_b1'], 'SAME'))
    return _conv(h, p[name + '_w2'], p[name + '_b2'], 'VALID')


def _roi_head(x, p, name):
    h = _conv(x, p[name + '_w1'], p[name + '_b1'], 'SAME')
    h = (h - p[name + '_bn_m'][None, :, None, None]) / jnp.sqrt(p[name + '_bn_v'][None, :, None, None] + 1e-5)
    h = h * p[name + '_bn_g'][None, :, None, None] + p[name + '_bn_b'][None, :, None, None]
    h = jax.nn.relu(h)
    h = jnp.mean(h, axis=(2, 3), keepdims=True)
    return _conv(h, p[name + '_w2'], p[name + '_b2'], 'VALID')


def _select_topk(heat, K):
    b, c, hh, ww = heat.shape
    flat = heat.reshape(b, c, hh * ww)
    s_all, i_all = jax.lax.top_k(flat, K)
    scores, inds = jax.lax.top_k(s_all.reshape(b, c * K), K)
    clses = inds // K
    inds_all = jnp.take_along_axis(i_all.reshape(b, c * K), inds, axis=1)
    return scores, inds_all, clses


def _bilinear(img, xs, ys):
    x0 = jnp.floor(xs)
    y0 = jnp.floor(ys)
    wx = xs - x0
    wy = ys - y0
    x0i = jnp.clip(x0.astype(jnp.int32), 0, W - 1)
    x1i = jnp.clip(x0.astype(jnp.int32) + 1, 0, W - 1)
    y0i = jnp.clip(y0.astype(jnp.int32), 0, H - 1)
    y1i = jnp.clip(y0.astype(jnp.int32) + 1, 0, H - 1)
    Ia = img[:, y0i, x0i]
    Ib = img[:, y0i, x1i]
    Ic = img[:, y1i, x0i]
    Id = img[:, y1i, x1i]
    return Ia * (1 - wx) * (1 - wy) + Ib * wx * (1 - wy) + Ic * (1 - wx) * wy + Id * wx * wy


def _roi_align(feat, boxes):
    def one(box):
        bidx = box[0].astype(jnp.int32)
        g = (jnp.arange(7, dtype=jnp.float32) + 0.5) / 7.0
        xs = box[1] + g * (box[3] - box[1])
        ys = box[2] + g * (box[4] - box[2])
        xg, yg = jnp.meshgrid(xs, ys)
        return _bilinear(feat[bidx], xg - 0.5, yg - 0.5)
    return jax.vmap(one)(boxes)


def _project(calib, pts):
    cu = calib[:, 0, 2]
    cv = calib[:, 1, 2]
    fu = calib[:, 0, 0]
    fv = calib[:, 1, 1]
    bx = calib[:, 0, 3] / (-fu)
    by = calib[:, 1, 3] / (-fv)
    x = (pts[:, 0] - cu) * pts[:, 2] / fu + bx
    y = (pts[:, 1] - cv) * pts[:, 2] / fv + by
    return jnp.stack([x, y, pts[:, 2]], -1)


def kernel(features, calib, coord_range, params):
    p = params
    heatmap, offset_2d, size_2d = _dense_heads(features, p)
    hm_nms = _nms_pallas(heatmap)
    scores, inds, clses = _select_topk(hm_nms, KDET)
    xg, yg = jnp.meshgrid(jnp.arange(W, dtype=jnp.float32), jnp.arange(H, dtype=jnp.float32))
    coord_map = jnp.broadcast_to(jnp.stack([xg, yg], 0)[None], (B, 2, H, W))
    center = coord_map + offset_2d
    bmaps = jnp.concatenate([center - size_2d / 2.0, center + size_2d / 2.0], 1)
    bids = jnp.broadcast_to(jnp.arange(B, dtype=jnp.float32)[:, None, None, None], (B, 1, H, W))
    bmaps = jnp.concatenate([bids, bmaps], 1)
    bm = bmaps.reshape(B, 5, H * W).transpose(0, 2, 1)
    box = jnp.take_along_axis(bm, inds[:, :, None], axis=1).reshape(B * KDET, 5)
    cls_ids = clses.reshape(B * KDET)
    roi_feat = _roi_align(features, box)
    bidx = box[:, 0].astype(jnp.int32)
    cr = coord_range[bidx]
    sx = cr[:, 1, 0] - cr[:, 0, 0]
    ox = cr[:, 0, 0]
    sy = cr[:, 1, 1] - cr[:, 0, 1]
    oy = cr[:, 0, 1]
    box_s = jnp.stack([box[:, 0], box[:, 1] / W * sx + ox, box[:, 2] / H * sy + oy,
                       box[:, 3] / W * sx + ox, box[:, 4] / H * sy + oy], -1)
    roi_calib = calib[bidx]
    N = B * KDET
    ones = jnp.ones((N, 1), dtype=jnp.float32)
    p1 = _project(roi_calib, jnp.concatenate([box_s[:, 1:3], ones], -1))[:, :2]
    p2 = _project(roi_calib, jnp.concatenate([box_s[:, 3:5], ones], -1))[:, :2]
    cic = jnp.concatenate([box_s[:, 0:1], p1, p2], -1)
    t = jnp.arange(7, dtype=jnp.float32) / 6.0
    cx = cic[:, 1:2] + t[None, :] * (cic[:, 3:4] - cic[:, 1:2])
    cy = cic[:, 2:3] + t[None, :] * (cic[:, 4:5] - cic[:, 2:3])
    coord_maps = jnp.concatenate([
        jnp.broadcast_to(cx[:, None, None, :], (N, 1, 7, 7)),
        jnp.broadcast_to(cy[:, None, :, None], (N, 1, 7, 7))], 1)
    cls_hot = jax.nn.one_hot(cls_ids, NUM_CLASS, dtype=jnp.float32)
    roi_in = jnp.concatenate([roi_feat, coord_maps,
                              jnp.broadcast_to(cls_hot[:, :, None, None], (N, NUM_CLASS, 7, 7))], 1)
    box2d_h = jnp.clip(box_s[:, 4] - box_s[:, 2], 1.0, None)
    s3d = _roi_head(roi_in, p, 's3d')[:, :, 0, 0]
    h3d_log_std = s3d[:, 3:4]
    size_3d = p['mean_size'][cls_ids] + s3d[:, :3]
    depth_geo = size_3d[:, 0] / box2d_h * roi_calib[:, 0, 0]
    dnet = _roi_head(roi_in, p, 'dep')[:, :, 0, 0]
    dgls = (h3d_log_std[:, 0] + 2.0 * (jnp.log(roi_calib[:, 0, 0]) - jnp.log(box2d_h)))[:, None]
    dnls = jax.nn.logsumexp(jnp.concatenate([dnet[:, 1:2], dgls], -1), axis=-1, keepdims=True)
    depth = jnp.concatenate([1.0 / (jax.nn.sigmoid(dnet[:, 0:1]) + 1e-6) - 1.0 + depth_geo[:, None], dnls], -1)
    heading = _roi_head(roi_in, p, 'hd')[:, :, 0, 0]
    offset_3d = _roi_head(roi_in, p, 'o3d')[:, :, 0, 0]
    return heatmap, offset_2d, size_2d, heading, depth, offset_3d, size_3d
